# Initial kernel scaffold; baseline (speedup 1.0000x reference)
import jax
import jax.numpy as jnp
from jax.experimental import pallas as pl


def kernel(Y, Ind, T, W):
    raise NotImplementedError("write your pallas kernel here")


# speedup vs baseline: 1.0936x; 1.0936x over previous
import functools

import jax
import jax.numpy as jnp
from jax import lax
from jax.experimental import pallas as pl
from jax.experimental.pallas import tpu as pltpu

HEADS = 16
HEAD_DIM = 128
OUT_DIM = 1024
BATCH = 4
KTOK = 512
TTOK = 4096
ROWS = HEADS * KTOK


def _mm_body(y_ref, w_ref, x_ref):
    x_ref[0, 0] = jnp.dot(y_ref[0, 0], w_ref[0],
                          preferred_element_type=jnp.float32)


def _tc_einsum(Y, W):
    return pl.pallas_call(
        _mm_body,
        grid=(BATCH, HEADS),
        in_specs=[
            pl.BlockSpec((1, 1, KTOK, HEAD_DIM), lambda b, e: (b, e, 0, 0)),
            pl.BlockSpec((1, HEAD_DIM, OUT_DIM), lambda b, e: (e, 0, 0)),
        ],
        out_specs=pl.BlockSpec((1, 1, KTOK, OUT_DIM), lambda b, e: (b, e, 0, 0)),
        out_shape=jax.ShapeDtypeStruct((BATCH, HEADS, KTOK, OUT_DIM),
                                       jnp.float32),
    )(Y, W)


def kernel(Y, Ind, T, W):
    X = _tc_einsum(Y, W).reshape(BATCH, ROWS, OUT_DIM)
    idx = jnp.mod(Ind.reshape(BATCH, ROWS).astype(jnp.int32),
                  jnp.asarray(T, jnp.int32))
    out = jnp.zeros((BATCH, TTOK, OUT_DIM), dtype=Y.dtype)
    bidx = jnp.arange(BATCH)[:, None]
    return out.at[bidx, idx].add(X)
